# baseline (speedup 1.0000x reference)
import functools

import jax
import jax.numpy as jnp
from jax import lax
from jax.experimental import pallas as pl
from jax.experimental.pallas import tpu as pltpu
from jax.experimental.pallas import tpu_sc as plsc

B = 16384
EMB = 32
EPS = 1e-5

NC = 2
NS = 16
NW = NC * NS
BPW = B // NW
CHUNK = 128
NCHUNK = BPW // CHUNK


def _sc_gather_body(uid_ref, mid_ref, t_umf, t_mmf, t_umlp, t_mmlp,
                    o_umf, o_mmf, o_umlp, o_mmlp,
                    idx_u, idx_m, r_umf, r_mmf, r_umlp, r_mmlp, sem):
    wid = lax.axis_index("s") * NC + lax.axis_index("c")
    base = wid * BPW
    pltpu.sync_copy(uid_ref.at[wid], idx_u)
    pltpu.sync_copy(mid_ref.at[wid], idx_m)
    copies = []
    for j in range(NCHUNK):
        dst = pl.ds(j * CHUNK, CHUNK)
        copies.append(pltpu.async_copy(t_umf.at[idx_u.at[j]], r_umf.at[dst], sem))
        copies.append(pltpu.async_copy(t_mmf.at[idx_m.at[j]], r_mmf.at[dst], sem))
        copies.append(pltpu.async_copy(t_umlp.at[idx_u.at[j]], r_umlp.at[dst], sem))
        copies.append(pltpu.async_copy(t_mmlp.at[idx_m.at[j]], r_mmlp.at[dst], sem))
    for c in copies:
        c.wait()
    out_slc = pl.ds(base, BPW)
    pltpu.sync_copy(r_umf, o_umf.at[out_slc])
    pltpu.sync_copy(r_mmf, o_mmf.at[out_slc])
    pltpu.sync_copy(r_umlp, o_umlp.at[out_slc])
    pltpu.sync_copy(r_mmlp, o_mmlp.at[out_slc])


@functools.partial(jax.jit, static_argnums=())
def _sc_gather(uid2d, mid2d, t_umf, t_mmf, t_umlp, t_mmlp):
    mesh = plsc.VectorSubcoreMesh(core_axis_name="c", subcore_axis_name="s")
    rows = jax.ShapeDtypeStruct((B, EMB), jnp.float32)
    return pl.kernel(
        _sc_gather_body,
        out_type=(rows, rows, rows, rows),
        mesh=mesh,
        compiler_params=pltpu.CompilerParams(use_tc_tiling_on_sc=False),
        scratch_types=(
            pltpu.VMEM((NCHUNK, CHUNK), jnp.int32),
            pltpu.VMEM((NCHUNK, CHUNK), jnp.int32),
            pltpu.VMEM((BPW, EMB), jnp.float32),
            pltpu.VMEM((BPW, EMB), jnp.float32),
            pltpu.VMEM((BPW, EMB), jnp.float32),
            pltpu.VMEM((BPW, EMB), jnp.float32),
            pltpu.SemaphoreType.DMA,
        ),
    )(uid2d, mid2d, t_umf, t_mmf, t_umlp, t_mmlp)


CH = 512
NCH = B // CH


def _dense_body(u_mf, m_mf, u_mlp, m_mlp,
                W0, b0, g0, be0, W1, b1, g1, be1, W2, b2, g2, be2, Wo, bo,
                out_ref, h0, h1, h2, mf_buf):
    f32 = jnp.float32
    hi = jax.lax.Precision.HIGHEST

    def dgen(W, x, dims):
        return lax.dot_general(W, x, (dims, ((), ())),
                               preferred_element_type=f32, precision=hi)

    W0v = W0[...]
    W0l, W0r = W0v[:, :EMB], W0v[:, EMB:]
    b0v, b1v, b2v = b0[...], b1[...], b2[...]

    for c in range(NCH):
        sl = pl.ds(c * CH, CH)
        um = u_mlp[sl, :]
        mm = m_mlp[sl, :]
        h = dgen(W0l, um, ((1,), (1,))) + dgen(W0r, mm, ((1,), (1,)))
        h = jnp.maximum(h + b0v[:, None], 0.0)
        h0[:, sl] = h
        mf = jnp.sum(u_mf[sl, :] * m_mf[sl, :], axis=1)
        mf_buf[0:1, sl] = mf.reshape(1, CH)

    def stats(href, g, be, n):
        hv = href[...]
        mean = jnp.sum(hv, axis=1) * (1.0 / B)
        var = jnp.sum(hv * hv, axis=1) * (1.0 / B) - mean * mean
        v = var + EPS
        r = lax.rsqrt(v)
        r = r * (1.5 - 0.5 * v * r * r)
        scale = g[...] * r
        shift = be[...] - mean * scale
        return scale, shift

    scale0, shift0 = stats(h0, g0, be0, 64)

    for c in range(NCH):
        sl = pl.ds(c * CH, CH)
        x = h0[:, sl] * scale0[:, None] + shift0[:, None]
        h = dgen(W1[...], x, ((1,), (0,)))
        h1[:, sl] = jnp.maximum(h + b1v[:, None], 0.0)

    scale1, shift1 = stats(h1, g1, be1, 32)

    for c in range(NCH):
        sl = pl.ds(c * CH, CH)
        x = h1[:, sl] * scale1[:, None] + shift1[:, None]
        h = dgen(W2[...], x, ((1,), (0,)))
        h2[:, sl] = jnp.maximum(h + b2v[:, None], 0.0)

    scale2, shift2 = stats(h2, g2, be2, 16)

    Wov = Wo[...]
    w_mlp = Wov[:, 1:].reshape(16, 1)
    w_mf = Wov[:, 0:1]
    bov = bo[...]
    for c in range(NCH):
        sl = pl.ds(c * CH, CH)
        x = h2[:, sl] * scale2[:, None] + shift2[:, None]
        acc = jnp.sum(x * w_mlp, axis=0).reshape(1, CH)
        out_ref[0:1, sl] = mf_buf[0:1, sl] * w_mf + acc + bov[None, :]


def _dense(u_mf, m_mf, u_mlp, m_mlp, weights):
    return pl.pallas_call(
        _dense_body,
        out_shape=jax.ShapeDtypeStruct((1, B), jnp.float32),
        scratch_shapes=[
            pltpu.VMEM((64, B), jnp.float32),
            pltpu.VMEM((32, B), jnp.float32),
            pltpu.VMEM((16, B), jnp.float32),
            pltpu.VMEM((1, B), jnp.float32),
        ],
    )(u_mf, m_mf, u_mlp, m_mlp, *weights)


def kernel(user_ids, movie_ids, ue_mf, me_mf, ue_mlp, me_mlp,
           W0, b0, g0, be0, W1, b1, g1, be1, W2, b2, g2, be2, Wo, bo):
    uid2d = user_ids.astype(jnp.int32).reshape(NW, NCHUNK, CHUNK)
    mid2d = movie_ids.astype(jnp.int32).reshape(NW, NCHUNK, CHUNK)
    u_mf, m_mf, u_mlp, m_mlp = _sc_gather(uid2d, mid2d, ue_mf, me_mf, ue_mlp, me_mlp)
    weights = (W0, b0, g0, be0, W1, b1, g1, be1, W2, b2, g2, be2, Wo, bo)
    out = _dense(u_mf, m_mf, u_mlp, m_mlp, weights)
    return out.reshape(B)
